# Initial kernel scaffold; baseline (speedup 1.0000x reference)
import jax
import jax.numpy as jnp
from jax.experimental import pallas as pl


def kernel(x, Adj_, W1, b1, W2, b2, W3, b3, Wp1, bp1, Wp2, bp2):
    raise NotImplementedError("write your pallas kernel here")


# speedup vs baseline: 1.0832x; 1.0832x over previous
import jax
import jax.numpy as jnp
from jax.experimental import pallas as pl

_BM = 400


def _g1_body(x_ref, w_ref, b_ref, g_ref):
    xb = x_ref[...].astype(jnp.bfloat16)
    g = jnp.dot(xb, w_ref[...], preferred_element_type=jnp.float32) + b_ref[...]
    g_ref[...] = g.astype(jnp.bfloat16)


def _layer1_body(adj_ref, g1_ref, w2_ref, b2_ref, adj16_ref, g2_ref):
    a16 = adj_ref[...].astype(jnp.bfloat16)
    adj16_ref[...] = a16
    h = jnp.dot(a16, g1_ref[...], preferred_element_type=jnp.float32)
    h = jnp.maximum(h, 0.0).astype(jnp.bfloat16)
    g2 = jnp.dot(h, w2_ref[...], preferred_element_type=jnp.float32) + b2_ref[...]
    g2_ref[...] = g2.astype(jnp.bfloat16)


def _layer2_body(adj16_ref, g2_ref, w3_ref, b3_ref, g3_ref):
    h = jnp.dot(adj16_ref[...], g2_ref[...], preferred_element_type=jnp.float32)
    h = jnp.maximum(h, 0.0).astype(jnp.bfloat16)
    g3 = jnp.dot(h, w3_ref[...], preferred_element_type=jnp.float32) + b3_ref[...]
    g3_ref[...] = g3.astype(jnp.bfloat16)


def _layer3_body(adj16_ref, g3_ref, wp1_ref, bp1_ref, wp2_ref, bp2_ref,
                 emb_ref, z_ref):
    emb = jnp.dot(adj16_ref[...], g3_ref[...], preferred_element_type=jnp.float32)
    emb_ref[...] = emb
    z1 = jnp.dot(emb.astype(jnp.bfloat16), wp1_ref[...],
                 preferred_element_type=jnp.float32) + bp1_ref[...]
    z1 = jnp.maximum(z1, 0.0).astype(jnp.bfloat16)
    z = jnp.dot(z1, wp2_ref[...], preferred_element_type=jnp.float32) + bp2_ref[...]
    z_ref[...] = z


def _full(shape):
    return pl.BlockSpec(shape, lambda i: (0, 0))


def kernel(x, Adj_, W1, b1, W2, b2, W3, b3, Wp1, bp1, Wp2, bp2):
    N, D = x.shape
    H = W1.shape[1]
    E = W3.shape[1]
    P = Wp2.shape[1]
    bm = _BM
    grid = (pl.cdiv(N, bm),)

    W1b, W2b, W3b, Wp1b, Wp2b = (
        w.astype(jnp.bfloat16) for w in (W1, W2, W3, Wp1, Wp2))
    b1r, b2r, b3r, bp1r, bp2r = (
        b.reshape(1, -1) for b in (b1, b2, b3, bp1, bp2))

    g1 = pl.pallas_call(
        _g1_body,
        grid=(1,),
        in_specs=[_full((N, D)), _full((D, H)), _full((1, H))],
        out_specs=_full((N, H)),
        out_shape=jax.ShapeDtypeStruct((N, H), jnp.bfloat16),
    )(x, W1b, b1r)

    adj16, g2 = pl.pallas_call(
        _layer1_body,
        grid=grid,
        in_specs=[pl.BlockSpec((bm, N), lambda i: (i, 0)),
                  _full((N, H)), _full((H, H)), _full((1, H))],
        out_specs=[pl.BlockSpec((bm, N), lambda i: (i, 0)),
                   pl.BlockSpec((bm, H), lambda i: (i, 0))],
        out_shape=[jax.ShapeDtypeStruct((N, N), jnp.bfloat16),
                   jax.ShapeDtypeStruct((N, H), jnp.bfloat16)],
    )(Adj_, g1, W2b, b2r)

    g3 = pl.pallas_call(
        _layer2_body,
        grid=grid,
        in_specs=[pl.BlockSpec((bm, N), lambda i: (i, 0)),
                  _full((N, H)), _full((H, H)), _full((1, H))],
        out_specs=pl.BlockSpec((bm, H), lambda i: (i, 0)),
        out_shape=jax.ShapeDtypeStruct((N, H), jnp.bfloat16),
    )(adj16, g2, W3b, b3r)

    emb, z = pl.pallas_call(
        _layer3_body,
        grid=grid,
        in_specs=[pl.BlockSpec((bm, N), lambda i: (i, 0)),
                  _full((N, H)), _full((E, P)), _full((1, P)),
                  _full((P, P)), _full((1, P))],
        out_specs=[pl.BlockSpec((bm, E), lambda i: (i, 0)),
                   pl.BlockSpec((bm, P), lambda i: (i, 0))],
        out_shape=[jax.ShapeDtypeStruct((N, E), jnp.float32),
                   jax.ShapeDtypeStruct((N, P), jnp.float32)],
    )(adj16, g3, Wp1b, bp1r, Wp2b, bp2r)

    return (z, emb)
